# baseline (speedup 1.0000x reference)
import functools

import jax
import jax.numpy as jnp
from jax import lax
from jax.experimental import pallas as pl
from jax.experimental.pallas import tpu as pltpu
from jax.experimental.pallas import tpu_sc as plsc

_NC, _NS, _LANES = 2, 16, 16
_CH = 128


def _build_rot(emb, R):
    vocab, d = emb.shape
    rows = 20000
    nb = vocab // rows

    def body(emb_ref, r_ref, out_ref):
        out_ref[...] = lax.dot_general(
            emb_ref[...], r_ref[...],
            (((1,), (1,)), ((), ())),
            preferred_element_type=jnp.float32)

    return pl.pallas_call(
        body,
        grid=(nb,),
        in_specs=[
            pl.BlockSpec((rows, d), lambda i: (i, 0)),
            pl.BlockSpec((d, d), lambda i: (0, 0)),
        ],
        out_specs=pl.BlockSpec((rows, d), lambda i: (i, 0)),
        out_shape=jax.ShapeDtypeStruct((vocab, d), jnp.float32),
    )(emb, R)


def _pass_body(keep_role, kpw):
    def body(table_hbm, tok_hbm, msk_hbm, out_hbm, tok_v, msk_v, cidx_v,
             cpos_v, rows_v, sem):
        wid = lax.axis_index("s") * _NC + lax.axis_index("c")
        pltpu.sync_copy(tok_hbm.at[wid], tok_v)
        pltpu.sync_copy(msk_hbm.at[wid], msk_v)
        base = wid * (kpw * _CH)
        lanes = lax.iota(jnp.int32, _LANES)

        def row_step(g, cnt):
            for j in range(_CH // _LANES):
                sl = pl.ds(j * _LANES, _LANES)
                t = tok_v[g, sl]
                m = msk_v[g, sl]
                keep = (m != 0) if keep_role else (m == 0)
                pc = plsc.cumsum(keep.astype(jnp.int32))
                dest = cnt + pc - 1
                drow = lax.shift_right_logical(dest, 7)
                dcol = jnp.bitwise_and(dest, _CH - 1)
                plsc.store_scatter(cidx_v, [drow, dcol], t, mask=keep)
                pos = base + g * _CH + j * _LANES + lanes
                plsc.store_scatter(cpos_v, [drow, dcol], pos, mask=keep)
                cnt = cnt + jnp.max(pc)
            return cnt

        cnt = lax.fori_loop(0, kpw, row_step, jnp.int32(0))
        full = lax.shift_right_logical(cnt, 7)
        rem = jnp.bitwise_and(cnt, _CH - 1)

        @pl.when(rem > 0)
        def _():
            i16 = cidx_v[0, pl.ds(0, _LANES)]
            p16 = cpos_v[0, pl.ds(0, _LANES)]
            big = jnp.int32(2147483647)
            i0 = jnp.min(jnp.where(lanes == 0, i16, big))
            p0 = jnp.min(jnp.where(lanes == 0, p16, big))
            for j in range(_CH // _LANES):
                sl = pl.ds(j * _LANES, _LANES)
                loc = j * _LANES + lanes
                ci = cidx_v[full, sl]
                cp = cpos_v[full, sl]
                cidx_v[full, sl] = jnp.where(loc >= rem, i0, ci)
                cpos_v[full, sl] = jnp.where(loc >= rem, p0, cp)

        nchunks = lax.shift_right_logical(cnt + _CH - 1, 7)

        def chunk(c, carry):
            pltpu.async_copy(table_hbm.at[cidx_v.at[c]], rows_v, sem).wait()
            pltpu.async_copy(rows_v, out_hbm.at[cpos_v.at[c]], sem).wait()
            return carry

        lax.fori_loop(0, nchunks, chunk, 0)

    return body


def _sc_mesh():
    return plsc.VectorSubcoreMesh(
        core_axis_name="c", subcore_axis_name="s",
        num_cores=_NC, num_subcores=_NS)


def _scratch(kpw, d):
    return [
        pltpu.VMEM((kpw, _CH), jnp.int32),
        pltpu.VMEM((kpw, _CH), jnp.int32),
        pltpu.VMEM((kpw, _CH), jnp.int32),
        pltpu.VMEM((kpw, _CH), jnp.int32),
        pltpu.VMEM((_CH, d), jnp.float32),
        pltpu.SemaphoreType.DMA,
    ]


def kernel(emb, R, token_ids, role_mask):
    vocab, d = emb.shape
    B, L = token_ids.shape
    n_tok = B * L
    nw = _NC * _NS
    kpw = n_tok // (nw * _CH)
    tok3 = token_ids.reshape(nw, kpw, _CH)
    msk3 = role_mask.astype(jnp.int32).reshape(nw, kpw, _CH)

    sc_params = pltpu.CompilerParams(needs_layout_passes=False)
    pass_a = functools.partial(
        pl.kernel,
        out_type=jax.ShapeDtypeStruct((n_tok, d), jnp.float32),
        mesh=_sc_mesh(), scratch_types=_scratch(kpw, d),
        compiler_params=sc_params,
    )(_pass_body(True, kpw))
    out = pass_a(emb, tok3, msk3)

    rot = _build_rot(emb, R)

    out_ref = jax.new_ref(out)
    pass_b = functools.partial(
        pl.kernel, mesh=_sc_mesh(), scratch_types=_scratch(kpw, d),
        compiler_params=sc_params,
    )(_pass_body(False, kpw))
    pass_b(rot, tok3, msk3, out_ref)

    return out_ref[...].reshape(B, L, d)


# speedup vs baseline: 1.0825x; 1.0825x over previous
import functools

import jax
import jax.numpy as jnp
from jax import lax
from jax.experimental import pallas as pl
from jax.experimental.pallas import tpu as pltpu
from jax.experimental.pallas import tpu_sc as plsc

_NC, _NS, _LANES = 2, 16, 16
_CH = 128


def _build_rot(emb, R):
    vocab, d = emb.shape
    rows = 20000
    nb = vocab // rows

    def body(emb_ref, r_ref, out_ref):
        out_ref[...] = lax.dot_general(
            emb_ref[...], r_ref[...],
            (((1,), (1,)), ((), ())),
            preferred_element_type=jnp.float32)

    return pl.pallas_call(
        body,
        grid=(nb,),
        in_specs=[
            pl.BlockSpec((rows, d), lambda i: (i, 0)),
            pl.BlockSpec((d, d), lambda i: (0, 0)),
        ],
        out_specs=pl.BlockSpec((rows, d), lambda i: (i, 0)),
        out_shape=jax.ShapeDtypeStruct((vocab, d), jnp.float32),
    )(emb, R)


def _pass_body(keep_role, kpw):
    def body(table_hbm, tok_hbm, msk_hbm, out_hbm, tok_v, msk_v, cidx_v,
             cpos_v, rows_v, g0, g1, s0, s1):
        gsems = (g0, g1)
        ssems = (s0, s1)
        wid = lax.axis_index("s") * _NC + lax.axis_index("c")
        pltpu.sync_copy(tok_hbm.at[wid], tok_v)
        pltpu.sync_copy(msk_hbm.at[wid], msk_v)
        base = wid * (kpw * _CH)
        lanes = lax.iota(jnp.int32, _LANES)

        def row_step(g, cnt):
            for j in range(_CH // _LANES):
                sl = pl.ds(j * _LANES, _LANES)
                t = tok_v[g, sl]
                m = msk_v[g, sl]
                keep = (m != 0) if keep_role else (m == 0)
                pc = plsc.cumsum(keep.astype(jnp.int32))
                dest = cnt + pc - 1
                drow = lax.shift_right_logical(dest, 7)
                dcol = jnp.bitwise_and(dest, _CH - 1)
                plsc.store_scatter(cidx_v, [drow, dcol], t, mask=keep)
                pos = base + g * _CH + j * _LANES + lanes
                plsc.store_scatter(cpos_v, [drow, dcol], pos, mask=keep)
                cnt = cnt + jnp.max(pc)
            return cnt

        cnt = lax.fori_loop(0, kpw, row_step, jnp.int32(0))
        full = lax.shift_right_logical(cnt, 7)
        rem = jnp.bitwise_and(cnt, _CH - 1)

        @pl.when(rem > 0)
        def _():
            i16 = cidx_v[0, pl.ds(0, _LANES)]
            p16 = cpos_v[0, pl.ds(0, _LANES)]
            big = jnp.int32(2147483647)
            i0 = jnp.min(jnp.where(lanes == 0, i16, big))
            p0 = jnp.min(jnp.where(lanes == 0, p16, big))
            for j in range(_CH // _LANES):
                sl = pl.ds(j * _LANES, _LANES)
                loc = j * _LANES + lanes
                ci = cidx_v[full, sl]
                cp = cpos_v[full, sl]
                cidx_v[full, sl] = jnp.where(loc >= rem, i0, ci)
                cpos_v[full, sl] = jnp.where(loc >= rem, p0, cp)

        nchunks = lax.shift_right_logical(cnt + _CH - 1, 7)

        def g_desc(c, b):
            return pltpu.make_async_copy(
                table_hbm.at[cidx_v.at[c]], rows_v.at[b], gsems[b])

        def s_desc(c, b):
            return pltpu.make_async_copy(
                rows_v.at[b], out_hbm.at[cpos_v.at[c]], ssems[b])

        @pl.when(nchunks > 0)
        def _():
            g_desc(0, 0).start()

        npairs = lax.shift_right_logical(nchunks + 1, 1)

        def pair(p, carry):
            for b in range(2):
                c = 2 * p + b

                @pl.when(c < nchunks)
                def _():
                    g_desc(c, b).wait()

                    @pl.when(c + 1 < nchunks)
                    def _():
                        @pl.when(c >= 1)
                        def _():
                            s_desc(0, 1 - b).wait()

                        g_desc(c + 1, 1 - b).start()

                    s_desc(c, b).start()
            return carry

        lax.fori_loop(0, npairs, pair, 0)

        @pl.when(nchunks > 1)
        def _():
            s_desc(0, 0).wait()
            s_desc(0, 1).wait()

        @pl.when(nchunks == 1)
        def _():
            s_desc(0, 0).wait()

    return body


def _sc_mesh():
    return plsc.VectorSubcoreMesh(
        core_axis_name="c", subcore_axis_name="s",
        num_cores=_NC, num_subcores=_NS)


def _scratch(kpw, d):
    return [
        pltpu.VMEM((kpw, _CH), jnp.int32),
        pltpu.VMEM((kpw, _CH), jnp.int32),
        pltpu.VMEM((kpw, _CH), jnp.int32),
        pltpu.VMEM((kpw, _CH), jnp.int32),
        pltpu.VMEM((2, _CH, d), jnp.float32),
        pltpu.SemaphoreType.DMA,
        pltpu.SemaphoreType.DMA,
        pltpu.SemaphoreType.DMA,
        pltpu.SemaphoreType.DMA,
    ]


def kernel(emb, R, token_ids, role_mask):
    vocab, d = emb.shape
    B, L = token_ids.shape
    n_tok = B * L
    nw = _NC * _NS
    kpw = n_tok // (nw * _CH)
    tok3 = token_ids.reshape(nw, kpw, _CH)
    msk3 = role_mask.astype(jnp.int32).reshape(nw, kpw, _CH)

    sc_params = pltpu.CompilerParams(needs_layout_passes=False)
    pass_a = functools.partial(
        pl.kernel,
        out_type=jax.ShapeDtypeStruct((n_tok, d), jnp.float32),
        mesh=_sc_mesh(), scratch_types=_scratch(kpw, d),
        compiler_params=sc_params,
    )(_pass_body(True, kpw))
    out = pass_a(emb, tok3, msk3)

    rot = _build_rot(emb, R)

    out_ref = jax.new_ref(out)
    pass_b = functools.partial(
        pl.kernel, mesh=_sc_mesh(), scratch_types=_scratch(kpw, d),
        compiler_params=sc_params,
    )(_pass_body(False, kpw))
    pass_b(rot, tok3, msk3, out_ref)

    return out_ref[...].reshape(B, L, d)


# speedup vs baseline: 1.1532x; 1.0653x over previous
import functools

import jax
import jax.numpy as jnp
from jax import lax
from jax.experimental import pallas as pl
from jax.experimental.pallas import tpu as pltpu
from jax.experimental.pallas import tpu_sc as plsc

_NC, _NS, _LANES = 2, 16, 16
_CH = 128
_NBUF = 4


def _build_rot(emb, R):
    vocab, d = emb.shape
    rows = 20000
    nb = vocab // rows

    def body(emb_ref, r_ref, out_ref):
        out_ref[...] = lax.dot_general(
            emb_ref[...], r_ref[...],
            (((1,), (1,)), ((), ())),
            preferred_element_type=jnp.float32)

    return pl.pallas_call(
        body,
        grid=(nb,),
        in_specs=[
            pl.BlockSpec((rows, d), lambda i: (i, 0)),
            pl.BlockSpec((d, d), lambda i: (0, 0)),
        ],
        out_specs=pl.BlockSpec((rows, d), lambda i: (i, 0)),
        out_shape=jax.ShapeDtypeStruct((vocab, d), jnp.float32),
    )(emb, R)


def _pass_body(keep_role, kpw):
    def body(table_hbm, tok_hbm, msk_hbm, out_hbm, tok_v, msk_v, cidx_v,
             cpos_v, rows_v, g0, g1, g2, g3, s0, s1, s2, s3):
        gsems = (g0, g1, g2, g3)
        ssems = (s0, s1, s2, s3)
        wid = lax.axis_index("s") * _NC + lax.axis_index("c")
        pltpu.sync_copy(tok_hbm.at[wid], tok_v)
        pltpu.sync_copy(msk_hbm.at[wid], msk_v)
        base = wid * (kpw * _CH)
        lanes = lax.iota(jnp.int32, _LANES)

        def row_step(g, cnt):
            for j in range(_CH // _LANES):
                sl = pl.ds(j * _LANES, _LANES)
                t = tok_v[g, sl]
                m = msk_v[g, sl]
                keep = (m != 0) if keep_role else (m == 0)
                pc = plsc.cumsum(keep.astype(jnp.int32))
                dest = cnt + pc - 1
                drow = lax.shift_right_logical(dest, 7)
                dcol = jnp.bitwise_and(dest, _CH - 1)
                plsc.store_scatter(cidx_v, [drow, dcol], t, mask=keep)
                pos = base + g * _CH + j * _LANES + lanes
                plsc.store_scatter(cpos_v, [drow, dcol], pos, mask=keep)
                cnt = cnt + jnp.max(pc)
            return cnt

        cnt = lax.fori_loop(0, kpw, row_step, jnp.int32(0))
        full = lax.shift_right_logical(cnt, 7)
        rem = jnp.bitwise_and(cnt, _CH - 1)

        @pl.when(rem > 0)
        def _():
            i16 = cidx_v[0, pl.ds(0, _LANES)]
            p16 = cpos_v[0, pl.ds(0, _LANES)]
            big = jnp.int32(2147483647)
            i0 = jnp.min(jnp.where(lanes == 0, i16, big))
            p0 = jnp.min(jnp.where(lanes == 0, p16, big))
            for j in range(_CH // _LANES):
                sl = pl.ds(j * _LANES, _LANES)
                loc = j * _LANES + lanes
                ci = cidx_v[full, sl]
                cp = cpos_v[full, sl]
                cidx_v[full, sl] = jnp.where(loc >= rem, i0, ci)
                cpos_v[full, sl] = jnp.where(loc >= rem, p0, cp)

        nchunks = lax.shift_right_logical(cnt + _CH - 1, 7)

        def g_desc(c, b):
            return pltpu.make_async_copy(
                table_hbm.at[cidx_v.at[c]], rows_v.at[b], gsems[b])

        def s_desc(c, b):
            return pltpu.make_async_copy(
                rows_v.at[b], out_hbm.at[cpos_v.at[c]], ssems[b])

        @pl.when(nchunks > 0)
        def _():
            g_desc(0, 0).start()

        @pl.when(nchunks > 1)
        def _():
            g_desc(1, 1).start()

        ngroups = lax.shift_right_logical(nchunks + 3, 2)

        def group(p, carry):
            for b in range(_NBUF):
                c = _NBUF * p + b

                @pl.when(c < nchunks)
                def _():
                    g_desc(c, b).wait()
                    nb = (b + 2) % _NBUF

                    @pl.when(c + 2 < nchunks)
                    def _():
                        @pl.when(c >= 2)
                        def _():
                            s_desc(0, nb).wait()

                        g_desc(c + 2, nb).start()

                    s_desc(c, b).start()
            return carry

        lax.fori_loop(0, ngroups, group, 0)

        for b in range(_NBUF):
            cond = ((nchunks >= 1) & (jnp.bitwise_and(nchunks - 1, 3) == b)) | (
                (nchunks >= 2) & (jnp.bitwise_and(nchunks - 2, 3) == b))

            @pl.when(cond)
            def _():
                s_desc(0, b).wait()

    return body


def _sc_mesh():
    return plsc.VectorSubcoreMesh(
        core_axis_name="c", subcore_axis_name="s",
        num_cores=_NC, num_subcores=_NS)


def _scratch(kpw, d):
    return [
        pltpu.VMEM((kpw, _CH), jnp.int32),
        pltpu.VMEM((kpw, _CH), jnp.int32),
        pltpu.VMEM((kpw, _CH), jnp.int32),
        pltpu.VMEM((kpw, _CH), jnp.int32),
        pltpu.VMEM((_NBUF, _CH, d), jnp.float32),
    ] + [pltpu.SemaphoreType.DMA] * (2 * _NBUF)


def kernel(emb, R, token_ids, role_mask):
    vocab, d = emb.shape
    B, L = token_ids.shape
    n_tok = B * L
    nw = _NC * _NS
    kpw = n_tok // (nw * _CH)
    tok3 = token_ids.reshape(nw, kpw, _CH)
    msk3 = role_mask.astype(jnp.int32).reshape(nw, kpw, _CH)

    sc_params = pltpu.CompilerParams(needs_layout_passes=False)
    pass_a = functools.partial(
        pl.kernel,
        out_type=jax.ShapeDtypeStruct((n_tok, d), jnp.float32),
        mesh=_sc_mesh(), scratch_types=_scratch(kpw, d),
        compiler_params=sc_params,
    )(_pass_body(True, kpw))
    out = pass_a(emb, tok3, msk3)

    rot = _build_rot(emb, R)

    out_ref = jax.new_ref(out)
    pass_b = functools.partial(
        pl.kernel, mesh=_sc_mesh(), scratch_types=_scratch(kpw, d),
        compiler_params=sc_params,
    )(_pass_body(False, kpw))
    pass_b(rot, tok3, msk3, out_ref)

    return out_ref[...].reshape(B, L, d)


# speedup vs baseline: 1.1763x; 1.0200x over previous
import functools

import jax
import jax.numpy as jnp
from jax import lax
from jax.experimental import pallas as pl
from jax.experimental.pallas import tpu as pltpu
from jax.experimental.pallas import tpu_sc as plsc

_NC, _NS, _LANES = 2, 16, 16
_CH = 128
_NBUF = 4


def _build_rot(emb, R):
    vocab, d = emb.shape
    rows = 20000
    nb = vocab // rows

    def body(emb_ref, r_ref, out_ref):
        out_ref[...] = lax.dot_general(
            emb_ref[...], r_ref[...],
            (((1,), (1,)), ((), ())),
            preferred_element_type=jnp.float32)

    return pl.pallas_call(
        body,
        grid=(nb,),
        in_specs=[
            pl.BlockSpec((rows, d), lambda i: (i, 0)),
            pl.BlockSpec((d, d), lambda i: (0, 0)),
        ],
        out_specs=pl.BlockSpec((rows, d), lambda i: (i, 0)),
        out_shape=jax.ShapeDtypeStruct((vocab, d), jnp.float32),
    )(emb, R)


def _pass_body(keep_role, kpw):
    def body(table_hbm, tok_hbm, msk_hbm, out_hbm, tok_v, msk_v, cidx_v,
             cpos_v, rows_v, g0, g1, g2, g3, s0, s1, s2, s3):
        gsems = (g0, g1, g2, g3)
        ssems = (s0, s1, s2, s3)
        wid = lax.axis_index("s") * _NC + lax.axis_index("c")
        pltpu.sync_copy(tok_hbm.at[wid], tok_v)
        pltpu.sync_copy(msk_hbm.at[wid], msk_v)
        base = wid * (kpw * _CH)
        lanes = lax.iota(jnp.int32, _LANES)

        def row_step(g, cnt):
            for j in range(_CH // _LANES):
                sl = pl.ds(j * _LANES, _LANES)
                t = tok_v[g, sl]
                m = msk_v[g, sl]
                keep = (m != 0) if keep_role else (m == 0)
                pc = plsc.cumsum(keep.astype(jnp.int32))
                dest = cnt + pc - 1
                drow = lax.shift_right_logical(dest, 7)
                dcol = jnp.bitwise_and(dest, _CH - 1)
                plsc.store_scatter(cidx_v, [drow, dcol], t, mask=keep)
                pos = base + g * _CH + j * _LANES + lanes
                plsc.store_scatter(cpos_v, [drow, dcol], pos, mask=keep)
                cnt = cnt + pc[_LANES - 1]
            return cnt

        cnt = lax.fori_loop(0, kpw, row_step, jnp.int32(0))
        full = lax.shift_right_logical(cnt, 7)
        rem = jnp.bitwise_and(cnt, _CH - 1)

        @pl.when(rem > 0)
        def _():
            i16 = cidx_v[0, pl.ds(0, _LANES)]
            p16 = cpos_v[0, pl.ds(0, _LANES)]
            i0 = i16[0]
            p0 = p16[0]
            for j in range(_CH // _LANES):
                sl = pl.ds(j * _LANES, _LANES)
                loc = j * _LANES + lanes
                ci = cidx_v[full, sl]
                cp = cpos_v[full, sl]
                cidx_v[full, sl] = jnp.where(loc >= rem, i0, ci)
                cpos_v[full, sl] = jnp.where(loc >= rem, p0, cp)

        nchunks = lax.shift_right_logical(cnt + _CH - 1, 7)

        def g_desc(c, b):
            return pltpu.make_async_copy(
                table_hbm.at[cidx_v.at[c]], rows_v.at[b], gsems[b])

        def s_desc(c, b):
            return pltpu.make_async_copy(
                rows_v.at[b], out_hbm.at[cpos_v.at[c]], ssems[b])

        @pl.when(nchunks > 0)
        def _():
            g_desc(0, 0).start()

        @pl.when(nchunks > 1)
        def _():
            g_desc(1, 1).start()

        @pl.when(nchunks > 2)
        def _():
            g_desc(2, 2).start()

        ngroups = lax.shift_right_logical(nchunks + 3, 2)

        def group(p, carry):
            for b in range(_NBUF):
                c = _NBUF * p + b

                @pl.when(c < nchunks)
                def _():
                    g_desc(c, b).wait()
                    nb = (b + 3) % _NBUF

                    @pl.when(c + 3 < nchunks)
                    def _():
                        @pl.when(c >= 1)
                        def _():
                            s_desc(0, nb).wait()

                        g_desc(c + 3, nb).start()

                    s_desc(c, b).start()
            return carry

        lax.fori_loop(0, ngroups, group, 0)

        for b in range(_NBUF):
            @pl.when(nchunks > b)
            def _():
                s_desc(0, b).wait()

    return body


def _sc_mesh():
    return plsc.VectorSubcoreMesh(
        core_axis_name="c", subcore_axis_name="s",
        num_cores=_NC, num_subcores=_NS)


def _scratch(kpw, d):
    return [
        pltpu.VMEM((kpw, _CH), jnp.int32),
        pltpu.VMEM((kpw, _CH), jnp.int32),
        pltpu.VMEM((kpw, _CH), jnp.int32),
        pltpu.VMEM((kpw, _CH), jnp.int32),
        pltpu.VMEM((_NBUF, _CH, d), jnp.float32),
    ] + [pltpu.SemaphoreType.DMA] * (2 * _NBUF)


def kernel(emb, R, token_ids, role_mask):
    vocab, d = emb.shape
    B, L = token_ids.shape
    n_tok = B * L
    nw = _NC * _NS
    kpw = n_tok // (nw * _CH)
    tok3 = token_ids.reshape(nw, kpw, _CH)
    msk3 = role_mask.astype(jnp.int32).reshape(nw, kpw, _CH)

    sc_params = pltpu.CompilerParams(needs_layout_passes=False)
    pass_a = functools.partial(
        pl.kernel,
        out_type=jax.ShapeDtypeStruct((n_tok, d), jnp.float32),
        mesh=_sc_mesh(), scratch_types=_scratch(kpw, d),
        compiler_params=sc_params,
    )(_pass_body(True, kpw))
    out = pass_a(emb, tok3, msk3)

    rot = _build_rot(emb, R)

    out_ref = jax.new_ref(out)
    pass_b = functools.partial(
        pl.kernel, mesh=_sc_mesh(), scratch_types=_scratch(kpw, d),
        compiler_params=sc_params,
    )(_pass_body(False, kpw))
    pass_b(rot, tok3, msk3, out_ref)

    return out_ref[...].reshape(B, L, d)


# speedup vs baseline: 1.1953x; 1.0161x over previous
import functools

import jax
import jax.numpy as jnp
from jax import lax
from jax.experimental import pallas as pl
from jax.experimental.pallas import tpu as pltpu
from jax.experimental.pallas import tpu_sc as plsc

_NC, _NS, _LANES = 2, 16, 16
_CH = 128
_NBUF = 4


def _build_rot(emb, R):
    vocab, d = emb.shape
    rows = 20000
    nb = vocab // rows

    def body(emb_ref, r_ref, out_ref):
        out_ref[...] = lax.dot_general(
            emb_ref[...], r_ref[...],
            (((1,), (1,)), ((), ())),
            preferred_element_type=jnp.float32)

    return pl.pallas_call(
        body,
        grid=(nb,),
        in_specs=[
            pl.BlockSpec((rows, d), lambda i: (i, 0)),
            pl.BlockSpec((d, d), lambda i: (0, 0)),
        ],
        out_specs=pl.BlockSpec((rows, d), lambda i: (i, 0)),
        out_shape=jax.ShapeDtypeStruct((vocab, d), jnp.float32),
    )(emb, R)


def _pass_body(keep_role, kpw, rpw, L):
    def body(table_hbm, tok_hbm, msk_hbm, out_hbm, tok_v, msk_v, cidx_v,
             cpos_v, rows_v, g0, g1, g2, g3, s0, s1, s2, s3):
        gsems = (g0, g1, g2, g3)
        ssems = (s0, s1, s2, s3)
        wid = lax.axis_index("s") * _NC + lax.axis_index("c")
        pltpu.sync_copy(tok_hbm.at[pl.ds(wid * rpw, rpw)], tok_v)
        pltpu.sync_copy(msk_hbm.at[pl.ds(wid * rpw, rpw)], msk_v)
        base = wid * (kpw * _CH)
        lanes = lax.iota(jnp.int32, _LANES)

        nfull = L // _LANES
        cols = ([j * _LANES for j in range(nfull)]
                + ([L - _LANES] if L % _LANES else []))

        def row_step(r, cnt):
            for i, col in enumerate(cols):
                sl = pl.ds(col, _LANES)
                t = tok_v[r, sl]
                m = msk_v[r, sl]
                keep = (m != 0) if keep_role else (m == 0)
                if i == nfull:
                    keep = keep & (lanes >= _LANES - (L % _LANES))
                pc = plsc.cumsum(keep.astype(jnp.int32))
                dest = cnt + pc - 1
                drow = lax.shift_right_logical(dest, 7)
                dcol = jnp.bitwise_and(dest, _CH - 1)
                plsc.store_scatter(cidx_v, [drow, dcol], t, mask=keep)
                pos = base + r * L + col + lanes
                plsc.store_scatter(cpos_v, [drow, dcol], pos, mask=keep)
                cnt = cnt + pc[_LANES - 1]
            return cnt

        cnt = lax.fori_loop(0, rpw, row_step, jnp.int32(0))
        full = lax.shift_right_logical(cnt, 7)
        rem = jnp.bitwise_and(cnt, _CH - 1)

        @pl.when(rem > 0)
        def _():
            i16 = cidx_v[0, pl.ds(0, _LANES)]
            p16 = cpos_v[0, pl.ds(0, _LANES)]
            i0 = i16[0]
            p0 = p16[0]
            for j in range(_CH // _LANES):
                sl = pl.ds(j * _LANES, _LANES)
                loc = j * _LANES + lanes
                ci = cidx_v[full, sl]
                cp = cpos_v[full, sl]
                cidx_v[full, sl] = jnp.where(loc >= rem, i0, ci)
                cpos_v[full, sl] = jnp.where(loc >= rem, p0, cp)

        nchunks = lax.shift_right_logical(cnt + _CH - 1, 7)

        def g_desc(c, b):
            return pltpu.make_async_copy(
                table_hbm.at[cidx_v.at[c]], rows_v.at[b], gsems[b])

        def s_desc(c, b):
            return pltpu.make_async_copy(
                rows_v.at[b], out_hbm.at[cpos_v.at[c]], ssems[b])

        @pl.when(nchunks > 0)
        def _():
            g_desc(0, 0).start()

        @pl.when(nchunks > 1)
        def _():
            g_desc(1, 1).start()

        @pl.when(nchunks > 2)
        def _():
            g_desc(2, 2).start()

        ngroups = lax.shift_right_logical(nchunks + 3, 2)

        def group(p, carry):
            for b in range(_NBUF):
                c = _NBUF * p + b

                @pl.when(c < nchunks)
                def _():
                    g_desc(c, b).wait()
                    nb = (b + 3) % _NBUF

                    @pl.when(c + 3 < nchunks)
                    def _():
                        @pl.when(c >= 1)
                        def _():
                            s_desc(0, nb).wait()

                        g_desc(c + 3, nb).start()

                    s_desc(c, b).start()
            return carry

        lax.fori_loop(0, ngroups, group, 0)

        for b in range(_NBUF):
            @pl.when(nchunks > b)
            def _():
                s_desc(0, b).wait()

    return body


def _sc_mesh():
    return plsc.VectorSubcoreMesh(
        core_axis_name="c", subcore_axis_name="s",
        num_cores=_NC, num_subcores=_NS)


def _scratch(kpw, rpw, L, d):
    return [
        pltpu.VMEM((rpw, L), jnp.int32),
        pltpu.VMEM((rpw, L), jnp.int32),
        pltpu.VMEM((kpw, _CH), jnp.int32),
        pltpu.VMEM((kpw, _CH), jnp.int32),
        pltpu.VMEM((_NBUF, _CH, d), jnp.float32),
    ] + [pltpu.SemaphoreType.DMA] * (2 * _NBUF)


def kernel(emb, R, token_ids, role_mask):
    vocab, d = emb.shape
    B, L = token_ids.shape
    n_tok = B * L
    nw = _NC * _NS
    kpw = -(-n_tok // (nw * _CH))
    rpw = B // nw
    msk_i = role_mask.astype(jnp.int32)

    sc_params = pltpu.CompilerParams(needs_layout_passes=False)
    pass_a = functools.partial(
        pl.kernel,
        out_type=jax.ShapeDtypeStruct((n_tok, d), jnp.float32),
        mesh=_sc_mesh(), scratch_types=_scratch(kpw, rpw, L, d),
        compiler_params=sc_params,
    )(_pass_body(True, kpw, rpw, L))
    out = pass_a(emb, token_ids, msk_i)

    rot = _build_rot(emb, R)

    out_ref = jax.new_ref(out)
    pass_b = functools.partial(
        pl.kernel, mesh=_sc_mesh(), scratch_types=_scratch(kpw, rpw, L, d),
        compiler_params=sc_params,
    )(_pass_body(False, kpw, rpw, L))
    pass_b(rot, token_ids, msk_i, out_ref)

    return out_ref[...].reshape(B, L, d)


# speedup vs baseline: 1.2368x; 1.0347x over previous
import functools

import jax
import jax.numpy as jnp
from jax import lax
from jax.experimental import pallas as pl
from jax.experimental.pallas import tpu as pltpu
from jax.experimental.pallas import tpu_sc as plsc

_NC, _NS, _LANES = 2, 16, 16
_CH = 128
_NBUF = 4


def _build_rot(emb, R):
    vocab, d = emb.shape
    rows = 20000
    nb = vocab // rows

    def body(emb_ref, r_ref, out_ref):
        out_ref[...] = lax.dot_general(
            emb_ref[...], r_ref[...],
            (((1,), (1,)), ((), ())),
            preferred_element_type=jnp.float32)

    return pl.pallas_call(
        body,
        grid=(nb,),
        in_specs=[
            pl.BlockSpec((rows, d), lambda i: (i, 0)),
            pl.BlockSpec((d, d), lambda i: (0, 0)),
        ],
        out_specs=pl.BlockSpec((rows, d), lambda i: (i, 0)),
        out_shape=jax.ShapeDtypeStruct((vocab, d), jnp.float32),
    )(emb, R)


def _pad_tail(idx_v, pos_v, cnt, lanes):
    full = lax.shift_right_logical(cnt, 7)
    rem = jnp.bitwise_and(cnt, _CH - 1)

    @pl.when(rem > 0)
    def _():
        i0 = idx_v[0, pl.ds(0, _LANES)][0]
        p0 = pos_v[0, pl.ds(0, _LANES)][0]
        for j in range(_CH // _LANES):
            sl = pl.ds(j * _LANES, _LANES)
            loc = j * _LANES + lanes
            ci = idx_v[full, sl]
            cp = pos_v[full, sl]
            idx_v[full, sl] = jnp.where(loc >= rem, i0, ci)
            pos_v[full, sl] = jnp.where(loc >= rem, p0, cp)


def _run_chunks(table_hbm, out_hbm, cidx_v, cpos_v, rows_v, gsems, ssems, cnt):
    nchunks = lax.shift_right_logical(cnt + _CH - 1, 7)

    def g_desc(c, b):
        return pltpu.make_async_copy(
            table_hbm.at[cidx_v.at[c]], rows_v.at[b], gsems[b])

    def s_desc(c, b):
        return pltpu.make_async_copy(
            rows_v.at[b], out_hbm.at[cpos_v.at[c]], ssems[b])

    for k in range(_NBUF - 1):
        @pl.when(nchunks > k)
        def _():
            g_desc(k, k).start()

    ngroups = lax.shift_right_logical(nchunks + _NBUF - 1, 2)

    def group(p, carry):
        for b in range(_NBUF):
            c = _NBUF * p + b

            @pl.when(c < nchunks)
            def _():
                g_desc(c, b).wait()
                nb = (b + 3) % _NBUF

                @pl.when(c + 3 < nchunks)
                def _():
                    @pl.when(c >= 1)
                    def _():
                        s_desc(0, nb).wait()

                    g_desc(c + 3, nb).start()

                s_desc(c, b).start()
        return carry

    lax.fori_loop(0, ngroups, group, 0)

    for b in range(_NBUF):
        @pl.when(nchunks > b)
        def _():
            s_desc(0, b).wait()


def _pass_a_body(kpw, rpw, L):
    def body(table_hbm, tok_hbm, msk_hbm,
             out_hbm, didx_hbm, dpos_hbm,
             tok_v, msk_v, cidx_v, cpos_v, didx_v, dpos_v, rows_v,
             g0, g1, g2, g3, s0, s1, s2, s3):
        gsems = (g0, g1, g2, g3)
        ssems = (s0, s1, s2, s3)
        wid = lax.axis_index("s") * _NC + lax.axis_index("c")
        pltpu.sync_copy(tok_hbm.at[pl.ds(wid * rpw, rpw)], tok_v)
        pltpu.sync_copy(msk_hbm.at[pl.ds(wid * rpw, rpw)], msk_v)
        base = wid * (kpw * _CH)
        lanes = lax.iota(jnp.int32, _LANES)

        nfull = L // _LANES
        cols = ([j * _LANES for j in range(nfull)]
                + ([L - _LANES] if L % _LANES else []))

        def row_step(r, carry):
            cnt, dnt = carry
            for i, col in enumerate(cols):
                sl = pl.ds(col, _LANES)
                t = tok_v[r, sl]
                m = msk_v[r, sl]
                keep_r = m != 0
                keep_d = m == 0
                if i == nfull:
                    valid = lanes >= _LANES - (L % _LANES)
                    keep_r = keep_r & valid
                    keep_d = keep_d & valid
                pos = base + r * L + col + lanes

                pc = plsc.cumsum(keep_r.astype(jnp.int32))
                dest = cnt + pc - 1
                drow = lax.shift_right_logical(dest, 7)
                dcol = jnp.bitwise_and(dest, _CH - 1)
                plsc.store_scatter(cidx_v, [drow, dcol], t, mask=keep_r)
                plsc.store_scatter(cpos_v, [drow, dcol], pos, mask=keep_r)
                cnt = cnt + pc[_LANES - 1]

                qc = plsc.cumsum(keep_d.astype(jnp.int32))
                dest = dnt + qc - 1
                drow = lax.shift_right_logical(dest, 7)
                dcol = jnp.bitwise_and(dest, _CH - 1)
                plsc.store_scatter(didx_v, [drow, dcol], t, mask=keep_d)
                plsc.store_scatter(dpos_v, [drow, dcol], pos, mask=keep_d)
                dnt = dnt + qc[_LANES - 1]
            return cnt, dnt

        cnt, dnt = lax.fori_loop(
            0, rpw, row_step, (jnp.int32(0), jnp.int32(0)))
        _pad_tail(cidx_v, cpos_v, cnt, lanes)
        _pad_tail(didx_v, dpos_v, dnt, lanes)

        dpos_v[kpw, pl.ds(0, _LANES)] = jnp.full(
            (_LANES,), dnt, dtype=jnp.int32)
        pltpu.sync_copy(didx_v, didx_hbm.at[wid])
        pltpu.sync_copy(dpos_v, dpos_hbm.at[wid])

        _run_chunks(table_hbm, out_hbm, cidx_v, cpos_v, rows_v,
                    gsems, ssems, cnt)

    return body


def _pass_b_body(kpw):
    def body(table_hbm, didx_hbm, dpos_hbm, out_hbm,
             cidx_v, cpos_v, rows_v, g0, g1, g2, g3, s0, s1, s2, s3):
        gsems = (g0, g1, g2, g3)
        ssems = (s0, s1, s2, s3)
        wid = lax.axis_index("s") * _NC + lax.axis_index("c")
        pltpu.sync_copy(didx_hbm.at[wid], cidx_v)
        pltpu.sync_copy(dpos_hbm.at[wid], cpos_v)
        cnt = cpos_v[kpw, pl.ds(0, _LANES)][0]
        _run_chunks(table_hbm, out_hbm, cidx_v, cpos_v, rows_v,
                    gsems, ssems, cnt)

    return body


def _sc_mesh():
    return plsc.VectorSubcoreMesh(
        core_axis_name="c", subcore_axis_name="s",
        num_cores=_NC, num_subcores=_NS)


def kernel(emb, R, token_ids, role_mask):
    vocab, d = emb.shape
    B, L = token_ids.shape
    n_tok = B * L
    nw = _NC * _NS
    kpw = -(-n_tok // (nw * _CH))
    rpw = B // nw
    msk_i = role_mask.astype(jnp.int32)
    sems = [pltpu.SemaphoreType.DMA] * (2 * _NBUF)

    sc_params = pltpu.CompilerParams(needs_layout_passes=False)
    pass_a = functools.partial(
        pl.kernel,
        out_type=(
            jax.ShapeDtypeStruct((n_tok, d), jnp.float32),
            jax.ShapeDtypeStruct((nw, kpw, _CH), jnp.int32),
            jax.ShapeDtypeStruct((nw, kpw + 1, _CH), jnp.int32),
        ),
        mesh=_sc_mesh(),
        scratch_types=[
            pltpu.VMEM((rpw, L), jnp.int32),
            pltpu.VMEM((rpw, L), jnp.int32),
            pltpu.VMEM((kpw, _CH), jnp.int32),
            pltpu.VMEM((kpw, _CH), jnp.int32),
            pltpu.VMEM((kpw, _CH), jnp.int32),
            pltpu.VMEM((kpw + 1, _CH), jnp.int32),
            pltpu.VMEM((_NBUF, _CH, d), jnp.float32),
        ] + sems,
        compiler_params=sc_params,
    )(_pass_a_body(kpw, rpw, L))
    out, didx3, dpos3 = pass_a(emb, token_ids, msk_i)

    rot = _build_rot(emb, R)

    out_ref = jax.new_ref(out)
    pass_b = functools.partial(
        pl.kernel,
        mesh=_sc_mesh(),
        scratch_types=[
            pltpu.VMEM((kpw, _CH), jnp.int32),
            pltpu.VMEM((kpw + 1, _CH), jnp.int32),
            pltpu.VMEM((_NBUF, _CH, d), jnp.float32),
        ] + sems,
        compiler_params=sc_params,
    )(_pass_b_body(kpw))
    pass_b(rot, didx3, dpos3, out_ref)

    return out_ref[...].reshape(B, L, d)
